# baseline (speedup 1.0000x reference)
import functools

import jax
import jax.numpy as jnp
from jax import lax
from jax.experimental import pallas as pl
from jax.experimental.pallas import tpu as pltpu
from jax.experimental.pallas import tpu_sc as plsc

_NBINS = 1000
_EMB = 32
_LANES = 16
_STRIDE = 33


def _binT_body(vals_ref, ids_ref):
    v = vals_ref[...]
    vmin = jnp.min(v, axis=1, keepdims=True)
    vmax = jnp.max(v, axis=1, keepdims=True)
    lo = jnp.log1p(vmin)
    hi = jnp.log1p(vmax)
    x = (jnp.log1p(v) - lo) / (hi - lo)
    x = jnp.clip(x, 0.0, 1.0)
    ids_ref[...] = (x * float(_NBINS)).astype(jnp.int32)


def _compute_idsT(vals_t):
    return pl.pallas_call(
        _binT_body,
        out_shape=jax.ShapeDtypeStruct(vals_t.shape, jnp.int32),
    )(vals_t)


def _make_sc_gather(n_cols, batch):
    mesh = plsc.VectorSubcoreMesh(core_axis_name="c", subcore_axis_name="s")
    nc = mesh.num_cores
    nw = nc * mesh.num_subcores
    bpw = batch // nw
    assert bpw * nw == batch and bpw % 128 == 0
    slab = n_cols * bpw
    tbl_flat = (_NBINS + 1) * _STRIDE
    groups = bpw // _LANES

    @functools.partial(
        pl.kernel,
        out_type=jax.ShapeDtypeStruct((n_cols, _EMB, batch), jnp.float32),
        mesh=mesh,
        scratch_types=[
            pltpu.VMEM((tbl_flat,), jnp.float32),
            pltpu.VMEM((slab,), jnp.int32),
            pltpu.VMEM((_EMB, bpw), jnp.float32),
            pltpu.VMEM((_EMB, bpw), jnp.float32),
            pltpu.SemaphoreType.DMA,
            pltpu.SemaphoreType.DMA,
        ],
        compiler_params=pltpu.CompilerParams(needs_layout_passes=False),
    )
    def sc_gather(tbl_hbm, ids_hbm, out_hbm, tbl_v, idx_v, buf_a, buf_b,
                  osem_a, osem_b):
        wid = lax.axis_index("s") * nc + lax.axis_index("c")
        i0 = wid * bpw
        pltpu.sync_copy(tbl_hbm, tbl_v)
        pltpu.sync_copy(ids_hbm.at[pl.ds(wid * slab, slab)], idx_v)

        def fill(c, buf):
            @plsc.parallel_loop(0, groups, 1, unroll=4)
            def g_body(g):
                base = idx_v[pl.ds(c * bpw + _LANES * g, _LANES)] * _STRIDE
                for d in range(_EMB):
                    buf[d, pl.ds(_LANES * g, _LANES)] = plsc.load_gather(
                        tbl_v, [base + d])

        def issue_out(c, buf, osem):
            pltpu.async_copy(buf, out_hbm.at[c, :, pl.ds(i0, bpw)], osem)

        def drain_out(buf, osem):
            pltpu.make_async_copy(buf, out_hbm.at[0, :, pl.ds(i0, bpw)],
                                  osem).wait()

        fill(0, buf_a)
        issue_out(0, buf_a, osem_a)
        fill(1, buf_b)
        issue_out(1, buf_b, osem_b)

        def pair(p, carry):
            c = 2 * p
            drain_out(buf_a, osem_a)
            fill(c, buf_a)
            issue_out(c, buf_a, osem_a)
            drain_out(buf_b, osem_b)
            fill(c + 1, buf_b)
            issue_out(c + 1, buf_b, osem_b)
            return carry

        lax.fori_loop(1, n_cols // 2, pair, 0)
        drain_out(buf_a, osem_a)
        drain_out(buf_b, osem_b)

    return sc_gather, nw, bpw


def kernel(values, table):
    batch, n_cols = values.shape
    vals_t = jnp.transpose(values)
    ids_t = _compute_idsT(vals_t)
    sc_gather, nw, bpw = _make_sc_gather(n_cols, batch)
    ids_flat = jnp.transpose(
        ids_t.reshape(n_cols, nw, bpw), (1, 0, 2)).reshape(-1)
    table_ext = jnp.concatenate(
        [table, jnp.full((1, table.shape[1]), jnp.nan, table.dtype)], axis=0)
    table_pad = jnp.pad(table_ext, ((0, 0), (0, _STRIDE - _EMB)))
    tbl_flat = table_pad.reshape(-1)
    out = sc_gather(tbl_flat, ids_flat)
    return jnp.transpose(out, (2, 0, 1))


# speedup vs baseline: 1.6319x; 1.6319x over previous
import functools

import jax
import jax.numpy as jnp
from jax import lax
from jax.experimental import pallas as pl
from jax.experimental.pallas import tpu as pltpu
from jax.experimental.pallas import tpu_sc as plsc

_NBINS = 1000
_EMB = 32
_LANES = 16
_STRIDE = 33


def _binT_body(vals_ref, ids_ref):
    v = vals_ref[...]
    vmin = jnp.min(v, axis=1, keepdims=True)
    vmax = jnp.max(v, axis=1, keepdims=True)
    lo = jnp.log1p(vmin)
    hi = jnp.log1p(vmax)
    x = (jnp.log1p(v) - lo) / (hi - lo)
    x = jnp.clip(x, 0.0, 1.0)
    ids_ref[...] = (x * float(_NBINS)).astype(jnp.int32)


def _compute_idsT(vals_t):
    return pl.pallas_call(
        _binT_body,
        out_shape=jax.ShapeDtypeStruct(vals_t.shape, jnp.int32),
    )(vals_t)


def _make_sc_gather(n_cols, batch):
    mesh = plsc.VectorSubcoreMesh(core_axis_name="c", subcore_axis_name="s")
    nc = mesh.num_cores
    nw = nc * mesh.num_subcores
    bpw = batch // nw
    assert bpw * nw == batch and bpw % 128 == 0
    slab = n_cols * bpw
    tbl_flat = (_NBINS + 1) * _STRIDE
    groups = bpw // _LANES

    @functools.partial(
        pl.kernel,
        out_type=jax.ShapeDtypeStruct((n_cols, _EMB, batch), jnp.float32),
        mesh=mesh,
        scratch_types=[
            pltpu.VMEM((tbl_flat,), jnp.float32),
            pltpu.VMEM((slab,), jnp.int32),
            pltpu.VMEM((_EMB, bpw), jnp.float32),
            pltpu.VMEM((_EMB, bpw), jnp.float32),
            pltpu.SemaphoreType.DMA,
            pltpu.SemaphoreType.DMA,
        ],
        compiler_params=pltpu.CompilerParams(needs_layout_passes=False),
    )
    def sc_gather(tbl_hbm, ids_hbm, out_hbm, tbl_v, idx_v, buf_a, buf_b,
                  osem_a, osem_b):
        wid = lax.axis_index("s") * nc + lax.axis_index("c")
        i0 = wid * bpw
        pltpu.sync_copy(tbl_hbm, tbl_v)
        pltpu.sync_copy(ids_hbm.at[pl.ds(wid * slab, slab)], idx_v)

        def fill(c, buf):
            @plsc.parallel_loop(0, groups, 1, unroll=1)
            def g_body(g):
                base = idx_v[pl.ds(c * bpw + _LANES * g, _LANES)] * _STRIDE
                for d in range(_EMB):
                    buf[d, pl.ds(_LANES * g, _LANES)] = plsc.load_gather(
                        tbl_v, [base + d])

        def issue_out(c, buf, osem):
            pltpu.async_copy(buf, out_hbm.at[c, :, pl.ds(i0, bpw)], osem)

        def drain_out(buf, osem):
            pltpu.make_async_copy(buf, out_hbm.at[0, :, pl.ds(i0, bpw)],
                                  osem).wait()

        fill(0, buf_a)
        issue_out(0, buf_a, osem_a)
        fill(1, buf_b)
        issue_out(1, buf_b, osem_b)

        def pair(p, carry):
            c = 2 * p
            drain_out(buf_a, osem_a)
            fill(c, buf_a)
            issue_out(c, buf_a, osem_a)
            drain_out(buf_b, osem_b)
            fill(c + 1, buf_b)
            issue_out(c + 1, buf_b, osem_b)
            return carry

        lax.fori_loop(1, n_cols // 2, pair, 0)
        drain_out(buf_a, osem_a)
        drain_out(buf_b, osem_b)

    return sc_gather, nw, bpw


def kernel(values, table):
    batch, n_cols = values.shape
    vals_t = jnp.transpose(values)
    ids_t = _compute_idsT(vals_t)
    sc_gather, nw, bpw = _make_sc_gather(n_cols, batch)
    ids_flat = jnp.transpose(
        ids_t.reshape(n_cols, nw, bpw), (1, 0, 2)).reshape(-1)
    table_ext = jnp.concatenate(
        [table, jnp.full((1, table.shape[1]), jnp.nan, table.dtype)], axis=0)
    table_pad = jnp.pad(table_ext, ((0, 0), (0, _STRIDE - _EMB)))
    tbl_flat = table_pad.reshape(-1)
    out = sc_gather(tbl_flat, ids_flat)
    return jnp.transpose(out, (2, 0, 1))
